# Initial kernel scaffold; baseline (speedup 1.0000x reference)
import jax
import jax.numpy as jnp
from jax.experimental import pallas as pl


def kernel(features):
    raise NotImplementedError("write your pallas kernel here")


# speedup vs baseline: 7.6953x; 7.6953x over previous
import jax
import jax.numpy as jnp
import numpy as np
from jax.experimental import pallas as pl


_INPUT_DIM = 768
_TARGET_DIM = 32
_POOL = 2
_SS = 32
_PS = _SS // _POOL
_NPOOL = _PS * _PS


def _pool_matrix() -> np.ndarray:
    P = np.zeros((_NPOOL, _SS * _SS), dtype=np.float32)
    for R in range(_PS):
        for C in range(_PS):
            p = R * _PS + C
            for dr in range(_POOL):
                for dc in range(_POOL):
                    s = (_POOL * R + dr) * _SS + (_POOL * C + dc)
                    P[p, s] = 1.0 / (_POOL * _POOL)
    return P


def _select_matrix() -> np.ndarray:
    idx = np.linspace(0, _INPUT_DIM - 1, _TARGET_DIM).astype(np.int64)
    S = np.zeros((_INPUT_DIM, _TARGET_DIM), dtype=np.float32)
    S[idx, np.arange(_TARGET_DIM)] = 1.0
    return S


def _body(x_ref, p_ref, s_ref, o_ref):
    x = x_ref[0]
    sel = jnp.dot(x, s_ref[...], preferred_element_type=jnp.float32)
    o_ref[0] = jnp.dot(p_ref[...], sel, preferred_element_type=jnp.float32)


def kernel(features):
    b, spatial, c = features.shape
    P = jnp.asarray(_pool_matrix())
    S = jnp.asarray(_select_matrix())
    return pl.pallas_call(
        _body,
        grid=(b,),
        in_specs=[
            pl.BlockSpec((1, spatial, c), lambda i: (i, 0, 0)),
            pl.BlockSpec((_NPOOL, spatial), lambda i: (0, 0)),
            pl.BlockSpec((c, _TARGET_DIM), lambda i: (0, 0)),
        ],
        out_specs=pl.BlockSpec((1, _NPOOL, _TARGET_DIM), lambda i: (i, 0, 0)),
        out_shape=jax.ShapeDtypeStruct((b, _NPOOL, _TARGET_DIM), jnp.float32),
    )(features, P, S)
